# Initial kernel scaffold; baseline (speedup 1.0000x reference)
import jax
import jax.numpy as jnp
from jax.experimental import pallas as pl


def kernel(x, H, W1, b1, W2, b2):
    raise NotImplementedError("write your pallas kernel here")


# speedup vs baseline: 33.8686x; 33.8686x over previous
import functools

import jax
import jax.numpy as jnp
from jax import lax
from jax.experimental import pallas as pl
from jax.experimental.pallas import tpu as pltpu
from jax.experimental.pallas import tpu_sc as plsc

N = 10000
NDUM = 112
NP = N + NDUM
E = 320000
D = 16
IN_CH = 128

NC = 2
NS = 16
TILES = NC * NS
SB = 128
EPT = 10240
NSTREAM = EPT // SB
EP = TILES * EPT
EROWS = EP // SB
RPT = EROWS // TILES
SPT = NP // NS

_MESH = dict(core_axis_name="c", subcore_axis_name="s", num_cores=NC,
             num_subcores=NS)
_SC_PARAMS = pltpu.CompilerParams(use_tc_tiling_on_sc=False)


def _zero_fill(zrow_v):
    def body(i, _):
        zrow_v[i, :] = jnp.zeros((D,), jnp.float32)
        return 0
    lax.fori_loop(0, SPT, body, 0, unroll=8)


def _sc_pass_body(table_hbm, gidx_hbm, sidx_hbm, out_hbm,
                  table_s, acc_s, gidx_v, sidx_v, rows_v, zrow_v, sem):
    c = lax.axis_index("c")
    s = lax.axis_index("s")
    wid = c * NS + s
    _zero_fill(zrow_v)
    row0 = s * SPT
    pltpu.sync_copy(table_hbm.at[pl.ds(row0, SPT)], table_s.at[pl.ds(row0, SPT)])
    pltpu.sync_copy(zrow_v, acc_s.at[pl.ds(row0, SPT)])
    plsc.subcore_barrier()
    r0 = wid * RPT
    pltpu.sync_copy(gidx_hbm.at[pl.ds(r0, RPT)], gidx_v)
    pltpu.sync_copy(sidx_hbm.at[pl.ds(r0, RPT)], sidx_v)

    def step(j, _):
        pltpu.async_copy(table_s.at[gidx_v.at[j]], rows_v, sem).wait()
        pltpu.sync_copy(rows_v, acc_s.at[sidx_v.at[j]], add=True)
        return 0

    lax.fori_loop(0, NSTREAM, step, 0)
    plsc.subcore_barrier()
    pltpu.sync_copy(acc_s.at[pl.ds(row0, SPT)], out_hbm.at[c, pl.ds(row0, SPT)])


_sc_pass = pl.kernel(
    _sc_pass_body,
    out_type=jax.ShapeDtypeStruct((NC, NP, D), jnp.float32),
    mesh=plsc.VectorSubcoreMesh(**_MESH),
    scratch_types=[
        pltpu.VMEM_SHARED((NP, D), jnp.float32),
        pltpu.VMEM_SHARED((NP, D), jnp.float32),
        pltpu.VMEM((RPT, SB), jnp.int32),
        pltpu.VMEM((RPT, SB), jnp.int32),
        pltpu.VMEM((SB, D), jnp.float32),
        pltpu.VMEM((SPT, D), jnp.float32),
        pltpu.SemaphoreType.DMA,
    ],
    compiler_params=_SC_PARAMS,
    name="hgnn_sc_pass",
)


def _sc_deg_body(nidx_hbm, hidx_hbm, dn_hbm, de_hbm,
                 dn_s, de_s, nidx_v, hidx_v, ones_v, zrow_v):
    c = lax.axis_index("c")
    s = lax.axis_index("s")
    wid = c * NS + s
    _zero_fill(zrow_v)

    def ones_body(i, _):
        ones_v[i, :] = jnp.ones((D,), jnp.float32)
        return 0
    lax.fori_loop(0, SB, ones_body, 0, unroll=8)

    row0 = s * SPT
    pltpu.sync_copy(zrow_v, dn_s.at[pl.ds(row0, SPT)])
    pltpu.sync_copy(zrow_v, de_s.at[pl.ds(row0, SPT)])
    plsc.subcore_barrier()
    r0 = wid * RPT
    pltpu.sync_copy(nidx_hbm.at[pl.ds(r0, RPT)], nidx_v)
    pltpu.sync_copy(hidx_hbm.at[pl.ds(r0, RPT)], hidx_v)

    def step(j, _):
        pltpu.sync_copy(ones_v, dn_s.at[nidx_v.at[j]], add=True)
        pltpu.sync_copy(ones_v, de_s.at[hidx_v.at[j]], add=True)
        return 0

    lax.fori_loop(0, NSTREAM, step, 0)
    plsc.subcore_barrier()
    pltpu.sync_copy(dn_s.at[pl.ds(row0, SPT)], dn_hbm.at[c, pl.ds(row0, SPT)])
    pltpu.sync_copy(de_s.at[pl.ds(row0, SPT)], de_hbm.at[c, pl.ds(row0, SPT)])


_sc_deg = pl.kernel(
    _sc_deg_body,
    out_type=(jax.ShapeDtypeStruct((NC, NP, D), jnp.float32),
              jax.ShapeDtypeStruct((NC, NP, D), jnp.float32)),
    mesh=plsc.VectorSubcoreMesh(**_MESH),
    scratch_types=[
        pltpu.VMEM_SHARED((NP, D), jnp.float32),
        pltpu.VMEM_SHARED((NP, D), jnp.float32),
        pltpu.VMEM((RPT, SB), jnp.int32),
        pltpu.VMEM((RPT, SB), jnp.int32),
        pltpu.VMEM((SB, D), jnp.float32),
        pltpu.VMEM((SPT, D), jnp.float32),
    ],
    compiler_params=_SC_PARAMS,
    name="hgnn_sc_degrees",
)


def _row_mask(y):
    rows = lax.broadcasted_iota(jnp.int32, (NP, D), 0)
    return jnp.where(rows < N, y, 0.0)


def _tc1_body(x_ref, w_ref, b_ref, o_ref):
    y = jnp.dot(x_ref[...], w_ref[...], preferred_element_type=jnp.float32)
    o_ref[pl.ds(0, N), :] = y + b_ref[...]
    o_ref[pl.ds(N, NDUM), :] = jnp.zeros((NDUM, D), jnp.float32)


_tc1 = pl.pallas_call(
    _tc1_body,
    out_shape=jax.ShapeDtypeStruct((NP, D), jnp.float32),
    name="hgnn_tc_in_proj",
)


def _tc2_body(xep_ref, dnp_ref, dep_ref, xen_ref, rdn_ref, rde_ref):
    dn = dnp_ref[0] + dnp_ref[1]
    de = dep_ref[0] + dep_ref[1]
    rdn = jnp.where(dn > 0, 1.0 / dn, 0.0)
    rde = jnp.where(de > 0, 1.0 / de, 0.0)
    xen_ref[...] = (xep_ref[0] + xep_ref[1]) * rde
    rdn_ref[...] = rdn
    rde_ref[...] = rde


_tc2 = pl.pallas_call(
    _tc2_body,
    out_shape=(jax.ShapeDtypeStruct((NP, D), jnp.float32),
               jax.ShapeDtypeStruct((NP, D), jnp.float32),
               jax.ShapeDtypeStruct((NP, D), jnp.float32)),
    name="hgnn_tc_norm_e",
)


def _tc3_body(xnp_ref, rdn_ref, w_ref, b_ref, o_ref):
    h = (xnp_ref[0] + xnp_ref[1]) * rdn_ref[...]
    h = jnp.where(h >= 0, h, 0.01 * h)
    y = jnp.dot(h, w_ref[...], preferred_element_type=jnp.float32) + b_ref[...]
    o_ref[...] = _row_mask(y)


_tc3 = pl.pallas_call(
    _tc3_body,
    out_shape=jax.ShapeDtypeStruct((NP, D), jnp.float32),
    name="hgnn_tc_mid",
)


def _tc4_body(xep_ref, rde_ref, o_ref):
    o_ref[...] = (xep_ref[0] + xep_ref[1]) * rde_ref[...]


_tc4 = pl.pallas_call(
    _tc4_body,
    out_shape=jax.ShapeDtypeStruct((NP, D), jnp.float32),
    name="hgnn_tc_norm_e2",
)


def _tc5_body(xnp_ref, rdn_ref, o_ref):
    z = (xnp_ref[0] + xnp_ref[1]) * rdn_ref[...]
    z = z[0:N, :]
    m = jnp.max(z, axis=1, keepdims=True)
    zs = z - m
    lse = jnp.log(jnp.sum(jnp.exp(zs), axis=1, keepdims=True))
    o_ref[...] = zs - lse


_tc5 = pl.pallas_call(
    _tc5_body,
    out_shape=jax.ShapeDtypeStruct((N, D), jnp.float32),
    name="hgnn_tc_out",
)


def kernel(x, H, W1, b1, W2, b2):
    node_idx = H[0]
    hyedge_idx = H[1]
    pad = N + (jnp.arange(EP - E, dtype=jnp.int32) % NDUM)
    nidx = jnp.concatenate([node_idx, pad]).reshape(EROWS, SB)
    hidx = jnp.concatenate([hyedge_idx, pad]).reshape(EROWS, SB)

    b1r = b1.reshape(1, D)
    b2r = b2.reshape(1, D)

    dn_p, de_p = _sc_deg(nidx, hidx)

    tbl1 = _tc1(x, W1, b1r)
    xe_p = _sc_pass(tbl1, nidx, hidx)
    xen, rdn, rde = _tc2(xe_p, dn_p, de_p)
    xn_p = _sc_pass(xen, hidx, nidx)
    tbl2 = _tc3(xn_p, rdn, W2, b2r)
    xe2_p = _sc_pass(tbl2, nidx, hidx)
    xe2n = _tc4(xe2_p, rde)
    xn2_p = _sc_pass(xe2n, hidx, nidx)
    return _tc5(xn2_p, rdn)


# speedup vs baseline: 48.4420x; 1.4303x over previous
import jax
import jax.numpy as jnp
from jax import lax
from jax.experimental import pallas as pl
from jax.experimental.pallas import tpu as pltpu
from jax.experimental.pallas import tpu_sc as plsc

N = 10000
NDUM = 112
NP = N + NDUM
E = 320000
D = 16
IN_CH = 128

NC = 2
NS = 16
TILES = NC * NS
SB = 128
EPT = 10240
NSTREAM = EPT // SB
EP = TILES * EPT
EROWS = EP // SB
RPT = EROWS // TILES
SPT = NP // NS

_MESH = dict(core_axis_name="c", subcore_axis_name="s", num_cores=NC,
             num_subcores=NS)
_SC_PARAMS = pltpu.CompilerParams(use_tc_tiling_on_sc=False)


def _fill_rows(ref, value):
    def body(i, _):
        ref[i, :] = jnp.full((D,), value, jnp.float32)
        return 0
    lax.fori_loop(0, ref.shape[0], body, 0, unroll=8)


def _edge_loop(table_s, acc_s, gidx_v, sidx_v, rows0_v, rows1_v, g0_sem,
               g1_sem):
    pltpu.async_copy(table_s.at[gidx_v.at[0]], rows0_v, g0_sem)

    def pair(p, _):
        jj = 2 * p
        pltpu.async_copy(table_s.at[gidx_v.at[jj + 1]], rows1_v, g1_sem)
        pltpu.make_async_copy(table_s.at[gidx_v.at[jj]], rows0_v, g0_sem).wait()
        pltpu.sync_copy(rows0_v, acc_s.at[sidx_v.at[jj]], add=True)

        @pl.when(jj + 2 < NSTREAM)
        def _():
            pltpu.async_copy(table_s.at[gidx_v.at[jj + 2]], rows0_v, g0_sem)

        pltpu.make_async_copy(table_s.at[gidx_v.at[jj + 1]], rows1_v,
                              g1_sem).wait()
        pltpu.sync_copy(rows1_v, acc_s.at[sidx_v.at[jj + 1]], add=True)
        return 0

    lax.fori_loop(0, NSTREAM // 2, pair, 0)


def _load_indices(gidx_hbm, sidx_hbm, gidx_v, sidx_v, wid):
    r0 = wid * RPT
    pltpu.sync_copy(gidx_hbm.at[pl.ds(r0, RPT)], gidx_v)
    pltpu.sync_copy(sidx_hbm.at[pl.ds(r0, RPT)], sidx_v)


def _sc_pass_body(table_hbm, gidx_hbm, sidx_hbm, out_hbm,
                  table_s, acc_s, gidx_v, sidx_v, rows0_v, rows1_v, zrow_v,
                  g0_sem, g1_sem):
    c = lax.axis_index("c")
    s = lax.axis_index("s")
    _fill_rows(zrow_v, 0.0)
    row0 = s * SPT
    pltpu.sync_copy(table_hbm.at[pl.ds(row0, SPT)], table_s.at[pl.ds(row0, SPT)])
    pltpu.sync_copy(zrow_v, acc_s.at[pl.ds(row0, SPT)])
    _load_indices(gidx_hbm, sidx_hbm, gidx_v, sidx_v, c * NS + s)
    plsc.subcore_barrier()
    _edge_loop(table_s, acc_s, gidx_v, sidx_v, rows0_v, rows1_v, g0_sem, g1_sem)
    plsc.subcore_barrier()
    pltpu.sync_copy(acc_s.at[pl.ds(row0, SPT)], out_hbm.at[c, pl.ds(row0, SPT)])


_sc_pass = pl.kernel(
    _sc_pass_body,
    out_type=jax.ShapeDtypeStruct((NC, NP, D), jnp.float32),
    mesh=plsc.VectorSubcoreMesh(**_MESH),
    scratch_types=[
        pltpu.VMEM_SHARED((NP, D), jnp.float32),
        pltpu.VMEM_SHARED((NP, D), jnp.float32),
        pltpu.VMEM((RPT, SB), jnp.int32),
        pltpu.VMEM((RPT, SB), jnp.int32),
        pltpu.VMEM((SB, D), jnp.float32),
        pltpu.VMEM((SB, D), jnp.float32),
        pltpu.VMEM((SPT, D), jnp.float32),
        pltpu.SemaphoreType.DMA,
        pltpu.SemaphoreType.DMA,
    ],
    compiler_params=_SC_PARAMS,
    name="hgnn_sc_pass",
)


def _sc_pass_norm_body(parts_hbm, rd_hbm, gidx_hbm, sidx_hbm, out_hbm,
                       table_s, acc_s, gidx_v, sidx_v, rows0_v, rows1_v,
                       st0_v, st1_v, st2_v, g0_sem, g1_sem):
    c = lax.axis_index("c")
    s = lax.axis_index("s")
    row0 = s * SPT
    slc = pl.ds(row0, SPT)
    pltpu.sync_copy(parts_hbm.at[0, slc], st0_v)
    pltpu.sync_copy(parts_hbm.at[1, slc], st1_v)
    pltpu.sync_copy(rd_hbm.at[slc], st2_v)

    def norm(i, _):
        st0_v[i, :] = (st0_v[i, :] + st1_v[i, :]) * st2_v[i, :]
        st1_v[i, :] = jnp.zeros((D,), jnp.float32)
        return 0

    lax.fori_loop(0, SPT, norm, 0, unroll=8)
    pltpu.sync_copy(st0_v, table_s.at[slc])
    pltpu.sync_copy(st1_v, acc_s.at[slc])
    _load_indices(gidx_hbm, sidx_hbm, gidx_v, sidx_v, c * NS + s)
    plsc.subcore_barrier()
    _edge_loop(table_s, acc_s, gidx_v, sidx_v, rows0_v, rows1_v, g0_sem, g1_sem)
    plsc.subcore_barrier()
    pltpu.sync_copy(acc_s.at[slc], out_hbm.at[c, slc])


_sc_pass_norm = pl.kernel(
    _sc_pass_norm_body,
    out_type=jax.ShapeDtypeStruct((NC, NP, D), jnp.float32),
    mesh=plsc.VectorSubcoreMesh(**_MESH),
    scratch_types=[
        pltpu.VMEM_SHARED((NP, D), jnp.float32),
        pltpu.VMEM_SHARED((NP, D), jnp.float32),
        pltpu.VMEM((RPT, SB), jnp.int32),
        pltpu.VMEM((RPT, SB), jnp.int32),
        pltpu.VMEM((SB, D), jnp.float32),
        pltpu.VMEM((SB, D), jnp.float32),
        pltpu.VMEM((SPT, D), jnp.float32),
        pltpu.VMEM((SPT, D), jnp.float32),
        pltpu.VMEM((SPT, D), jnp.float32),
        pltpu.SemaphoreType.DMA,
        pltpu.SemaphoreType.DMA,
    ],
    compiler_params=_SC_PARAMS,
    name="hgnn_sc_pass_norm",
)


def _sc_deg_body(nidx_hbm, hidx_hbm, dn_hbm, de_hbm,
                 dn_s, de_s, nidx_v, hidx_v, ones_v, zrow_v, a_sem, b_sem):
    c = lax.axis_index("c")
    s = lax.axis_index("s")
    _fill_rows(zrow_v, 0.0)
    _fill_rows(ones_v, 1.0)
    row0 = s * SPT
    pltpu.sync_copy(zrow_v, dn_s.at[pl.ds(row0, SPT)])
    pltpu.sync_copy(zrow_v, de_s.at[pl.ds(row0, SPT)])
    _load_indices(nidx_hbm, hidx_hbm, nidx_v, hidx_v, c * NS + s)
    plsc.subcore_barrier()

    def step(j, _):
        d1 = pltpu.async_copy(ones_v, dn_s.at[nidx_v.at[j]], a_sem, add=True)
        d2 = pltpu.async_copy(ones_v, de_s.at[hidx_v.at[j]], b_sem, add=True)
        d1.wait()
        d2.wait()
        return 0

    lax.fori_loop(0, NSTREAM, step, 0)
    plsc.subcore_barrier()
    pltpu.sync_copy(dn_s.at[pl.ds(row0, SPT)], dn_hbm.at[c, pl.ds(row0, SPT)])
    pltpu.sync_copy(de_s.at[pl.ds(row0, SPT)], de_hbm.at[c, pl.ds(row0, SPT)])


_sc_deg = pl.kernel(
    _sc_deg_body,
    out_type=(jax.ShapeDtypeStruct((NC, NP, D), jnp.float32),
              jax.ShapeDtypeStruct((NC, NP, D), jnp.float32)),
    mesh=plsc.VectorSubcoreMesh(**_MESH),
    scratch_types=[
        pltpu.VMEM_SHARED((NP, D), jnp.float32),
        pltpu.VMEM_SHARED((NP, D), jnp.float32),
        pltpu.VMEM((RPT, SB), jnp.int32),
        pltpu.VMEM((RPT, SB), jnp.int32),
        pltpu.VMEM((SB, D), jnp.float32),
        pltpu.VMEM((SPT, D), jnp.float32),
        pltpu.SemaphoreType.DMA,
        pltpu.SemaphoreType.DMA,
    ],
    compiler_params=_SC_PARAMS,
    name="hgnn_sc_degrees",
)


def _row_mask(y):
    rows = lax.broadcasted_iota(jnp.int32, (NP, D), 0)
    return jnp.where(rows < N, y, 0.0)


def _tc1_body(x_ref, w_ref, b_ref, o_ref):
    y = jnp.dot(x_ref[...], w_ref[...], preferred_element_type=jnp.float32)
    o_ref[pl.ds(0, N), :] = y + b_ref[...]
    o_ref[pl.ds(N, NDUM), :] = jnp.zeros((NDUM, D), jnp.float32)


_tc1 = pl.pallas_call(
    _tc1_body,
    out_shape=jax.ShapeDtypeStruct((NP, D), jnp.float32),
    name="hgnn_tc_in_proj",
)


def _tc_recip_body(dnp_ref, dep_ref, rdn_ref, rde_ref):
    dn = dnp_ref[0] + dnp_ref[1]
    de = dep_ref[0] + dep_ref[1]
    rdn_ref[...] = jnp.where(dn > 0, 1.0 / dn, 0.0)
    rde_ref[...] = jnp.where(de > 0, 1.0 / de, 0.0)


_tc_recip = pl.pallas_call(
    _tc_recip_body,
    out_shape=(jax.ShapeDtypeStruct((NP, D), jnp.float32),
               jax.ShapeDtypeStruct((NP, D), jnp.float32)),
    name="hgnn_tc_recip",
)


def _tc3_body(xnp_ref, rdn_ref, w_ref, b_ref, o_ref):
    h = (xnp_ref[0] + xnp_ref[1]) * rdn_ref[...]
    h = jnp.where(h >= 0, h, 0.01 * h)
    y = jnp.dot(h, w_ref[...], preferred_element_type=jnp.float32) + b_ref[...]
    o_ref[...] = _row_mask(y)


_tc3 = pl.pallas_call(
    _tc3_body,
    out_shape=jax.ShapeDtypeStruct((NP, D), jnp.float32),
    name="hgnn_tc_mid",
)


def _tc5_body(xnp_ref, rdn_ref, o_ref):
    z = (xnp_ref[0] + xnp_ref[1]) * rdn_ref[...]
    z = z[0:N, :]
    m = jnp.max(z, axis=1, keepdims=True)
    zs = z - m
    lse = jnp.log(jnp.sum(jnp.exp(zs), axis=1, keepdims=True))
    o_ref[...] = zs - lse


_tc5 = pl.pallas_call(
    _tc5_body,
    out_shape=jax.ShapeDtypeStruct((N, D), jnp.float32),
    name="hgnn_tc_out",
)


def kernel(x, H, W1, b1, W2, b2):
    node_idx = H[0]
    hyedge_idx = H[1]
    pad = N + (jnp.arange(EP - E, dtype=jnp.int32) % NDUM)
    nidx = jnp.concatenate([node_idx, pad]).reshape(EROWS, SB)
    hidx = jnp.concatenate([hyedge_idx, pad]).reshape(EROWS, SB)

    b1r = b1.reshape(1, D)
    b2r = b2.reshape(1, D)

    dn_p, de_p = _sc_deg(nidx, hidx)
    rdn, rde = _tc_recip(dn_p, de_p)

    tbl1 = _tc1(x, W1, b1r)
    xe_p = _sc_pass(tbl1, nidx, hidx)
    xn_p = _sc_pass_norm(xe_p, rde, hidx, nidx)
    tbl2 = _tc3(xn_p, rdn, W2, b2r)
    xe2_p = _sc_pass(tbl2, nidx, hidx)
    xn2_p = _sc_pass_norm(xe2_p, rde, hidx, nidx)
    return _tc5(xn2_p, rdn)
